# baseline (speedup 1.0000x reference)
import functools

import jax
import jax.numpy as jnp
from jax import lax
from jax.experimental import pallas as pl
from jax.experimental.pallas import tpu as pltpu
from jax.experimental.pallas import tpu_sc as plsc

_B = 1024
_M = 16
_C = 128
_NNZ = 512
_NT = 34

_LANES = 16
_NW = 32
_CCHUNKS = _C // _LANES
_MPTR_PAD = 32

_SPLIT = 448
_BB = 128


def _sc_tensor_product(x1, x2, w, cg, p12, paw, mptr_pad):
    b_sc = x1.shape[0]
    bpw = b_sc // _NW
    mesh = plsc.VectorSubcoreMesh(core_axis_name="c", subcore_axis_name="s")

    @functools.partial(
        pl.kernel,
        mesh=mesh,
        out_type=jax.ShapeDtypeStruct((b_sc, _M, _C), jnp.float32),
        compiler_params=pltpu.CompilerParams(needs_layout_passes=False),
        scratch_types=[
            pltpu.SMEM((_NNZ,), jnp.int32),
            pltpu.SMEM((_NNZ,), jnp.int32),
            pltpu.SMEM((_NNZ,), jnp.float32),
            pltpu.SMEM((_MPTR_PAD,), jnp.int32),
            pltpu.VMEM((_NNZ,), jnp.int32),
            pltpu.VMEM((_NNZ,), jnp.int32),
            pltpu.VMEM((_NNZ,), jnp.float32),
            pltpu.VMEM((_MPTR_PAD,), jnp.int32),
            pltpu.VMEM((_M, _C), jnp.float32),
            pltpu.VMEM((_M, _C), jnp.float32),
            pltpu.VMEM((_NT, _C), jnp.float32),
            pltpu.VMEM((_M, _C), jnp.float32),
            pltpu.VMEM((_M, _C), jnp.float32),
            pltpu.VMEM((_NT, _C), jnp.float32),
            pltpu.VMEM((_M, _C), jnp.float32),
            pltpu.VMEM((_M, _C), jnp.float32),
            pltpu.SemaphoreType.DMA,
            pltpu.SemaphoreType.DMA,
            pltpu.SemaphoreType.DMA,
            pltpu.SemaphoreType.DMA,
        ],
    )
    def k(x1_hbm, x2_hbm, w_hbm, cg_hbm, p12_hbm, paw_hbm, mptr_hbm,
          out_hbm, p12_s, paw_s, cg_s, mptr_s, p12_b, paw_b, cg_b, mptr_b,
          x1_va, x2_va, w_va, x1_vb, x2_vb, w_vb, out_va, out_vb,
          sem_a, sem_b, sem_oa, sem_ob):
        wid = lax.axis_index("c") * 16 + lax.axis_index("s")

        pltpu.sync_copy(p12_hbm, p12_b)
        pltpu.sync_copy(paw_hbm, paw_b)
        pltpu.sync_copy(cg_hbm, cg_b)
        pltpu.sync_copy(mptr_hbm, mptr_b)

        @plsc.parallel_loop(0, _NNZ, _LANES)
        def fill_body(base):
            v12 = p12_b[pl.ds(base, _LANES)]
            vaw = paw_b[pl.ds(base, _LANES)]
            vcg = cg_b[pl.ds(base, _LANES)]
            for j in range(_LANES):
                p12_s[base + j] = v12[j]
                paw_s[base + j] = vaw[j]
                cg_s[base + j] = vcg[j]

        @plsc.parallel_loop(0, _MPTR_PAD, _LANES)
        def fill_mptr(base):
            vmp = mptr_b[pl.ds(base, _LANES)]
            for j in range(_LANES):
                mptr_s[base + j] = vmp[j]

        b0 = wid * bpw
        bufs = ((x1_va, x2_va, w_va, sem_a), (x1_vb, x2_vb, w_vb, sem_b))
        obufs = ((out_va, sem_oa), (out_vb, sem_ob))

        def start_copies(b, buf):
            x1d, x2d, wd, sem = buf
            pltpu.async_copy(x1_hbm.at[b], x1d, sem)
            pltpu.async_copy(x2_hbm.at[b], x2d, sem)
            pltpu.async_copy(w_hbm.at[b], wd, sem)

        def wait_copies(buf):
            x1d, x2d, wd, sem = buf
            pltpu.make_async_copy(x1_hbm.at[b0], x1d, sem).wait()
            pltpu.make_async_copy(x2_hbm.at[b0], x2d, sem).wait()
            pltpu.make_async_copy(w_hbm.at[b0], wd, sem).wait()

        def compute(b, i2, buf, obuf):
            x1d, x2d, wd, _ = buf
            out_v, sem_o = obuf

            @pl.when(i2 > 0)
            def _():
                pltpu.make_async_copy(out_v, out_hbm.at[b0], sem_o).wait()

            def seg_body(m, carry2):
                st = mptr_s[m]
                en = mptr_s[m + 1]
                zero = jnp.zeros((_LANES,), jnp.float32)
                init = (zero,) * _CCHUNKS

                @plsc.parallel_loop(st, en, 1, unroll=2, carry=init)
                def acc_fin(n, acc):
                    s12 = p12_s[n]
                    aws = paw_s[n]
                    cgs = cg_s[n]
                    o1 = s12 & 255
                    o2 = lax.shift_right_logical(s12, 8)
                    new = []
                    for kk in range(_CCHUNKS):
                        g1 = x1d[o1, pl.ds(kk * _LANES, _LANES)]
                        g2 = x2d[o2, pl.ds(kk * _LANES, _LANES)]
                        gw = wd[aws, pl.ds(kk * _LANES, _LANES)]
                        new.append(acc[kk] + g1 * g2 * gw * cgs)
                    return tuple(new)

                for kk in range(_CCHUNKS):
                    out_v[m, pl.ds(kk * _LANES, _LANES)] = acc_fin[kk]
                return carry2
            lax.fori_loop(0, _M, seg_body, 0)

            pltpu.async_copy(out_v, out_hbm.at[b], sem_o)

        start_copies(b0, bufs[0])

        def batch_pair(i2, carry):
            for par in range(2):
                i = i2 * 2 + par
                b = b0 + i
                buf = bufs[par]
                nxt = bufs[1 - par]
                wait_copies(buf)
                start_copies(b0 + jnp.minimum(i + 1, bpw - 1), nxt)
                compute(b, i2, buf, obufs[par])
            return carry
        lax.fori_loop(0, bpw // 2, batch_pair, 0)
        wait_copies(bufs[0])
        for out_v, sem_o in obufs:
            pltpu.make_async_copy(out_v, out_hbm.at[b0], sem_o).wait()

    return k(x1, x2, w, cg, p12, paw, mptr_pad)


def _tc_body(p12_ref, paw_ref, cg_ref, mptr_ref, x1_ref, x2_ref, w_ref,
             out_ref):
    for m in range(_M):
        st = mptr_ref[m]
        en = mptr_ref[m + 1]

        def e_body(n, acc):
            s12 = p12_ref[n]
            aws = paw_ref[n]
            cgs = cg_ref[n]
            o1 = s12 & 255
            o2 = lax.shift_right_logical(s12, 8)
            g1 = x1_ref[o1]
            g2 = x2_ref[o2]
            gw = w_ref[aws]
            return acc + g1 * g2 * gw * cgs

        acc = lax.fori_loop(st, en, e_body,
                            jnp.zeros((_BB, _C), jnp.float32))
        out_ref[m] = acc


def _tc_tensor_product(x1t, x2t, wt, cg, p12, paw, mptr):
    b_tc = x1t.shape[1]
    grid = (b_tc // _BB,)
    return pl.pallas_call(
        _tc_body,
        grid=grid,
        in_specs=[
            pl.BlockSpec(memory_space=pltpu.SMEM),
            pl.BlockSpec(memory_space=pltpu.SMEM),
            pl.BlockSpec(memory_space=pltpu.SMEM),
            pl.BlockSpec(memory_space=pltpu.SMEM),
            pl.BlockSpec((_M, _BB, _C), lambda i: (0, i, 0)),
            pl.BlockSpec((_M, _BB, _C), lambda i: (0, i, 0)),
            pl.BlockSpec((_NT, _BB, _C), lambda i: (0, i, 0)),
        ],
        out_specs=pl.BlockSpec((_M, _BB, _C), lambda i: (0, i, 0)),
        out_shape=jax.ShapeDtypeStruct((_M, b_tc, _C), jnp.float32),
        compiler_params=pltpu.CompilerParams(
            dimension_semantics=("arbitrary",)),
    )(p12, paw, cg, mptr, x1t, x2t, wt)


def kernel(x1, x2, weight, CG_vals, l_ind_M1M2, M1, M2, M_ptr_M1M2):
    p12 = M1 | (M2 << 8)
    mptr_pad = jnp.concatenate(
        [M_ptr_M1M2, jnp.zeros((_MPTR_PAD - _M - 1,), jnp.int32)])

    out_sc = _sc_tensor_product(
        x1[_SPLIT:], x2[_SPLIT:], weight[_SPLIT:],
        CG_vals, p12, l_ind_M1M2, mptr_pad)
    out_tct = _tc_tensor_product(
        jnp.swapaxes(x1[:_SPLIT], 0, 1),
        jnp.swapaxes(x2[:_SPLIT], 0, 1),
        jnp.swapaxes(weight[:_SPLIT], 0, 1),
        CG_vals, p12, l_ind_M1M2, M_ptr_M1M2)
    out_tc = jnp.swapaxes(out_tct, 0, 1)
    return jnp.concatenate([out_tc, out_sc], axis=0)


# speedup vs baseline: 1.0005x; 1.0005x over previous
import functools

import jax
import jax.numpy as jnp
from jax import lax
from jax.experimental import pallas as pl
from jax.experimental.pallas import tpu as pltpu
from jax.experimental.pallas import tpu_sc as plsc

_B = 1024
_M = 16
_C = 128
_NNZ = 512
_NT = 34

_LANES = 16
_NW = 32
_CCHUNKS = _C // _LANES
_MPTR_PAD = 32

_SPLIT = 448
_BB = 64


def _sc_tensor_product(x1, x2, w, cg, p12, paw, mptr_pad):
    b_sc = x1.shape[0]
    bpw = b_sc // _NW
    mesh = plsc.VectorSubcoreMesh(core_axis_name="c", subcore_axis_name="s")

    @functools.partial(
        pl.kernel,
        mesh=mesh,
        out_type=jax.ShapeDtypeStruct((b_sc, _M, _C), jnp.float32),
        compiler_params=pltpu.CompilerParams(needs_layout_passes=False),
        scratch_types=[
            pltpu.SMEM((_NNZ,), jnp.int32),
            pltpu.SMEM((_NNZ,), jnp.int32),
            pltpu.SMEM((_NNZ,), jnp.float32),
            pltpu.SMEM((_MPTR_PAD,), jnp.int32),
            pltpu.VMEM((_NNZ,), jnp.int32),
            pltpu.VMEM((_NNZ,), jnp.int32),
            pltpu.VMEM((_NNZ,), jnp.float32),
            pltpu.VMEM((_MPTR_PAD,), jnp.int32),
            pltpu.VMEM((_M, _C), jnp.float32),
            pltpu.VMEM((_M, _C), jnp.float32),
            pltpu.VMEM((_NT, _C), jnp.float32),
            pltpu.VMEM((_M, _C), jnp.float32),
            pltpu.VMEM((_M, _C), jnp.float32),
            pltpu.VMEM((_NT, _C), jnp.float32),
            pltpu.VMEM((_M, _C), jnp.float32),
            pltpu.VMEM((_M, _C), jnp.float32),
            pltpu.SemaphoreType.DMA,
            pltpu.SemaphoreType.DMA,
            pltpu.SemaphoreType.DMA,
            pltpu.SemaphoreType.DMA,
        ],
    )
    def k(x1_hbm, x2_hbm, w_hbm, cg_hbm, p12_hbm, paw_hbm, mptr_hbm,
          out_hbm, p12_s, paw_s, cg_s, mptr_s, p12_b, paw_b, cg_b, mptr_b,
          x1_va, x2_va, w_va, x1_vb, x2_vb, w_vb, out_va, out_vb,
          sem_a, sem_b, sem_oa, sem_ob):
        wid = lax.axis_index("c") * 16 + lax.axis_index("s")

        pltpu.sync_copy(p12_hbm, p12_b)
        pltpu.sync_copy(paw_hbm, paw_b)
        pltpu.sync_copy(cg_hbm, cg_b)
        pltpu.sync_copy(mptr_hbm, mptr_b)

        @plsc.parallel_loop(0, _NNZ, _LANES)
        def fill_body(base):
            v12 = p12_b[pl.ds(base, _LANES)]
            vaw = paw_b[pl.ds(base, _LANES)]
            vcg = cg_b[pl.ds(base, _LANES)]
            for j in range(_LANES):
                p12_s[base + j] = v12[j]
                paw_s[base + j] = vaw[j]
                cg_s[base + j] = vcg[j]

        @plsc.parallel_loop(0, _MPTR_PAD, _LANES)
        def fill_mptr(base):
            vmp = mptr_b[pl.ds(base, _LANES)]
            for j in range(_LANES):
                mptr_s[base + j] = vmp[j]

        b0 = wid * bpw
        bufs = ((x1_va, x2_va, w_va, sem_a), (x1_vb, x2_vb, w_vb, sem_b))
        obufs = ((out_va, sem_oa), (out_vb, sem_ob))

        def start_copies(b, buf):
            x1d, x2d, wd, sem = buf
            pltpu.async_copy(x1_hbm.at[b], x1d, sem)
            pltpu.async_copy(x2_hbm.at[b], x2d, sem)
            pltpu.async_copy(w_hbm.at[b], wd, sem)

        def wait_copies(buf):
            x1d, x2d, wd, sem = buf
            pltpu.make_async_copy(x1_hbm.at[b0], x1d, sem).wait()
            pltpu.make_async_copy(x2_hbm.at[b0], x2d, sem).wait()
            pltpu.make_async_copy(w_hbm.at[b0], wd, sem).wait()

        def compute(b, i2, buf, obuf):
            x1d, x2d, wd, _ = buf
            out_v, sem_o = obuf

            @pl.when(i2 > 0)
            def _():
                pltpu.make_async_copy(out_v, out_hbm.at[b0], sem_o).wait()

            def seg_body(m, carry2):
                st = mptr_s[m]
                en = mptr_s[m + 1]
                zero = jnp.zeros((_LANES,), jnp.float32)
                init = (zero,) * _CCHUNKS

                @plsc.parallel_loop(st, en, 1, unroll=2, carry=init)
                def acc_fin(n, acc):
                    s12 = p12_s[n]
                    aws = paw_s[n]
                    cgs = cg_s[n]
                    o1 = s12 & 255
                    o2 = lax.shift_right_logical(s12, 8)
                    new = []
                    for kk in range(_CCHUNKS):
                        g1 = x1d[o1, pl.ds(kk * _LANES, _LANES)]
                        g2 = x2d[o2, pl.ds(kk * _LANES, _LANES)]
                        gw = wd[aws, pl.ds(kk * _LANES, _LANES)]
                        new.append(acc[kk] + g1 * g2 * gw * cgs)
                    return tuple(new)

                for kk in range(_CCHUNKS):
                    out_v[m, pl.ds(kk * _LANES, _LANES)] = acc_fin[kk]
                return carry2
            lax.fori_loop(0, _M, seg_body, 0)

            pltpu.async_copy(out_v, out_hbm.at[b], sem_o)

        start_copies(b0, bufs[0])

        def batch_pair(i2, carry):
            for par in range(2):
                i = i2 * 2 + par
                b = b0 + i
                buf = bufs[par]
                nxt = bufs[1 - par]
                wait_copies(buf)
                start_copies(b0 + jnp.minimum(i + 1, bpw - 1), nxt)
                compute(b, i2, buf, obufs[par])
            return carry
        lax.fori_loop(0, bpw // 2, batch_pair, 0)
        wait_copies(bufs[0])
        for out_v, sem_o in obufs:
            pltpu.make_async_copy(out_v, out_hbm.at[b0], sem_o).wait()

    return k(x1, x2, w, cg, p12, paw, mptr_pad)


def _tc_body(p12_ref, paw_ref, cg_ref, mptr_ref, x1_ref, x2_ref, w_ref,
             out_ref):
    for m in range(_M):
        st = mptr_ref[m]
        en = mptr_ref[m + 1]

        def e_body(n, acc):
            s12 = p12_ref[n]
            aws = paw_ref[n]
            cgs = cg_ref[n]
            o1 = s12 & 255
            o2 = lax.shift_right_logical(s12, 8)
            g1 = x1_ref[o1]
            g2 = x2_ref[o2]
            gw = w_ref[aws]
            return acc + g1 * g2 * gw * cgs

        acc = lax.fori_loop(st, en, e_body,
                            jnp.zeros((_BB, _C), jnp.float32))
        out_ref[m] = acc


def _tc_tensor_product(x1t, x2t, wt, cg, p12, paw, mptr):
    b_tc = x1t.shape[1]
    grid = (b_tc // _BB,)
    return pl.pallas_call(
        _tc_body,
        grid=grid,
        in_specs=[
            pl.BlockSpec(memory_space=pltpu.SMEM),
            pl.BlockSpec(memory_space=pltpu.SMEM),
            pl.BlockSpec(memory_space=pltpu.SMEM),
            pl.BlockSpec(memory_space=pltpu.SMEM),
            pl.BlockSpec((_M, _BB, _C), lambda i: (0, i, 0)),
            pl.BlockSpec((_M, _BB, _C), lambda i: (0, i, 0)),
            pl.BlockSpec((_NT, _BB, _C), lambda i: (0, i, 0)),
        ],
        out_specs=pl.BlockSpec((_M, _BB, _C), lambda i: (0, i, 0)),
        out_shape=jax.ShapeDtypeStruct((_M, b_tc, _C), jnp.float32),
        compiler_params=pltpu.CompilerParams(
            dimension_semantics=("arbitrary",)),
    )(p12, paw, cg, mptr, x1t, x2t, wt)


def kernel(x1, x2, weight, CG_vals, l_ind_M1M2, M1, M2, M_ptr_M1M2):
    p12 = M1 | (M2 << 8)
    mptr_pad = jnp.concatenate(
        [M_ptr_M1M2, jnp.zeros((_MPTR_PAD - _M - 1,), jnp.int32)])

    out_sc = _sc_tensor_product(
        x1[_SPLIT:], x2[_SPLIT:], weight[_SPLIT:],
        CG_vals, p12, l_ind_M1M2, mptr_pad)
    out_tct = _tc_tensor_product(
        jnp.swapaxes(x1[:_SPLIT], 0, 1),
        jnp.swapaxes(x2[:_SPLIT], 0, 1),
        jnp.swapaxes(weight[:_SPLIT], 0, 1),
        CG_vals, p12, l_ind_M1M2, M_ptr_M1M2)
    out_tc = jnp.swapaxes(out_tct, 0, 1)
    return jnp.concatenate([out_tc, out_sc], axis=0)


# speedup vs baseline: 1.0411x; 1.0405x over previous
import functools

import jax
import jax.numpy as jnp
from jax import lax
from jax.experimental import pallas as pl
from jax.experimental.pallas import tpu as pltpu
from jax.experimental.pallas import tpu_sc as plsc

_B = 1024
_M = 16
_C = 128
_NNZ = 512
_NT = 34

_LANES = 16
_NW = 32
_CCHUNKS = _C // _LANES
_MPTR_PAD = 32

_SPLIT = 512
_BB = 64


def _sc_tensor_product(x1, x2, w, cg, p12, paw, mptr_pad):
    b_sc = x1.shape[0]
    bpw = b_sc // _NW
    mesh = plsc.VectorSubcoreMesh(core_axis_name="c", subcore_axis_name="s")

    @functools.partial(
        pl.kernel,
        mesh=mesh,
        out_type=jax.ShapeDtypeStruct((b_sc, _M, _C), jnp.float32),
        compiler_params=pltpu.CompilerParams(needs_layout_passes=False),
        scratch_types=[
            pltpu.SMEM((_NNZ,), jnp.int32),
            pltpu.SMEM((_NNZ,), jnp.int32),
            pltpu.SMEM((_NNZ,), jnp.float32),
            pltpu.SMEM((_MPTR_PAD,), jnp.int32),
            pltpu.VMEM((_NNZ,), jnp.int32),
            pltpu.VMEM((_NNZ,), jnp.int32),
            pltpu.VMEM((_NNZ,), jnp.float32),
            pltpu.VMEM((_MPTR_PAD,), jnp.int32),
            pltpu.VMEM((_M, _C), jnp.float32),
            pltpu.VMEM((_M, _C), jnp.float32),
            pltpu.VMEM((_NT, _C), jnp.float32),
            pltpu.VMEM((_M, _C), jnp.float32),
            pltpu.VMEM((_M, _C), jnp.float32),
            pltpu.VMEM((_NT, _C), jnp.float32),
            pltpu.VMEM((_M, _C), jnp.float32),
            pltpu.VMEM((_M, _C), jnp.float32),
            pltpu.SemaphoreType.DMA,
            pltpu.SemaphoreType.DMA,
            pltpu.SemaphoreType.DMA,
            pltpu.SemaphoreType.DMA,
        ],
    )
    def k(x1_hbm, x2_hbm, w_hbm, cg_hbm, p12_hbm, paw_hbm, mptr_hbm,
          out_hbm, p12_s, paw_s, cg_s, mptr_s, p12_b, paw_b, cg_b, mptr_b,
          x1_va, x2_va, w_va, x1_vb, x2_vb, w_vb, out_va, out_vb,
          sem_a, sem_b, sem_oa, sem_ob):
        wid = lax.axis_index("c") * 16 + lax.axis_index("s")

        pltpu.sync_copy(p12_hbm, p12_b)
        pltpu.sync_copy(paw_hbm, paw_b)
        pltpu.sync_copy(cg_hbm, cg_b)
        pltpu.sync_copy(mptr_hbm, mptr_b)

        @plsc.parallel_loop(0, _NNZ, _LANES)
        def fill_body(base):
            v12 = p12_b[pl.ds(base, _LANES)]
            vaw = paw_b[pl.ds(base, _LANES)]
            vcg = cg_b[pl.ds(base, _LANES)]
            for j in range(_LANES):
                p12_s[base + j] = v12[j]
                paw_s[base + j] = vaw[j]
                cg_s[base + j] = vcg[j]

        @plsc.parallel_loop(0, _MPTR_PAD, _LANES)
        def fill_mptr(base):
            vmp = mptr_b[pl.ds(base, _LANES)]
            for j in range(_LANES):
                mptr_s[base + j] = vmp[j]

        b0 = wid * bpw
        bufs = ((x1_va, x2_va, w_va, sem_a), (x1_vb, x2_vb, w_vb, sem_b))
        obufs = ((out_va, sem_oa), (out_vb, sem_ob))

        def start_copies(b, buf):
            x1d, x2d, wd, sem = buf
            pltpu.async_copy(x1_hbm.at[b], x1d, sem)
            pltpu.async_copy(x2_hbm.at[b], x2d, sem)
            pltpu.async_copy(w_hbm.at[b], wd, sem)

        def wait_copies(buf):
            x1d, x2d, wd, sem = buf
            pltpu.make_async_copy(x1_hbm.at[b0], x1d, sem).wait()
            pltpu.make_async_copy(x2_hbm.at[b0], x2d, sem).wait()
            pltpu.make_async_copy(w_hbm.at[b0], wd, sem).wait()

        def compute(b, i2, buf, obuf):
            x1d, x2d, wd, _ = buf
            out_v, sem_o = obuf

            @pl.when(i2 > 0)
            def _():
                pltpu.make_async_copy(out_v, out_hbm.at[b0], sem_o).wait()

            def seg_body(m, carry2):
                st = mptr_s[m]
                en = mptr_s[m + 1]
                zero = jnp.zeros((_LANES,), jnp.float32)
                init = (zero,) * _CCHUNKS

                @plsc.parallel_loop(st, en, 1, unroll=2, carry=init)
                def acc_fin(n, acc):
                    s12 = p12_s[n]
                    aws = paw_s[n]
                    cgs = cg_s[n]
                    o1 = s12 & 255
                    o2 = lax.shift_right_logical(s12, 8)
                    new = []
                    for kk in range(_CCHUNKS):
                        g1 = x1d[o1, pl.ds(kk * _LANES, _LANES)]
                        g2 = x2d[o2, pl.ds(kk * _LANES, _LANES)]
                        gw = wd[aws, pl.ds(kk * _LANES, _LANES)]
                        new.append(acc[kk] + g1 * g2 * gw * cgs)
                    return tuple(new)

                for kk in range(_CCHUNKS):
                    out_v[m, pl.ds(kk * _LANES, _LANES)] = acc_fin[kk]
                return carry2
            lax.fori_loop(0, _M, seg_body, 0)

            pltpu.async_copy(out_v, out_hbm.at[b], sem_o)

        start_copies(b0, bufs[0])

        def batch_pair(i2, carry):
            for par in range(2):
                i = i2 * 2 + par
                b = b0 + i
                buf = bufs[par]
                nxt = bufs[1 - par]
                wait_copies(buf)
                start_copies(b0 + jnp.minimum(i + 1, bpw - 1), nxt)
                compute(b, i2, buf, obufs[par])
            return carry
        lax.fori_loop(0, bpw // 2, batch_pair, 0)
        wait_copies(bufs[0])
        for out_v, sem_o in obufs:
            pltpu.make_async_copy(out_v, out_hbm.at[b0], sem_o).wait()

    return k(x1, x2, w, cg, p12, paw, mptr_pad)


def _tc_body(p12_ref, paw_ref, cg_ref, mptr_ref, x1_ref, x2_ref, w_ref,
             out_ref):
    for m in range(_M):
        st = mptr_ref[m]
        en = mptr_ref[m + 1]

        def e_body(n, acc):
            s12 = p12_ref[n]
            aws = paw_ref[n]
            cgs = cg_ref[n]
            o1 = s12 & 255
            o2 = lax.shift_right_logical(s12, 8)
            g1 = x1_ref[o1]
            g2 = x2_ref[o2]
            gw = w_ref[aws]
            return acc + g1 * g2 * gw * cgs

        acc = lax.fori_loop(st, en, e_body,
                            jnp.zeros((_BB, _C), jnp.float32))
        out_ref[m] = acc


def _tc_tensor_product(x1t, x2t, wt, cg, p12, paw, mptr):
    b_tc = x1t.shape[1]
    grid = (b_tc // _BB,)
    return pl.pallas_call(
        _tc_body,
        grid=grid,
        in_specs=[
            pl.BlockSpec(memory_space=pltpu.SMEM),
            pl.BlockSpec(memory_space=pltpu.SMEM),
            pl.BlockSpec(memory_space=pltpu.SMEM),
            pl.BlockSpec(memory_space=pltpu.SMEM),
            pl.BlockSpec((_M, _BB, _C), lambda i: (0, i, 0)),
            pl.BlockSpec((_M, _BB, _C), lambda i: (0, i, 0)),
            pl.BlockSpec((_NT, _BB, _C), lambda i: (0, i, 0)),
        ],
        out_specs=pl.BlockSpec((_M, _BB, _C), lambda i: (0, i, 0)),
        out_shape=jax.ShapeDtypeStruct((_M, b_tc, _C), jnp.float32),
        compiler_params=pltpu.CompilerParams(
            dimension_semantics=("arbitrary",)),
    )(p12, paw, cg, mptr, x1t, x2t, wt)


def kernel(x1, x2, weight, CG_vals, l_ind_M1M2, M1, M2, M_ptr_M1M2):
    p12 = M1 | (M2 << 8)
    mptr_pad = jnp.concatenate(
        [M_ptr_M1M2, jnp.zeros((_MPTR_PAD - _M - 1,), jnp.int32)])

    out_sc = _sc_tensor_product(
        x1[_SPLIT:], x2[_SPLIT:], weight[_SPLIT:],
        CG_vals, p12, l_ind_M1M2, mptr_pad)
    out_tct = _tc_tensor_product(
        jnp.swapaxes(x1[:_SPLIT], 0, 1),
        jnp.swapaxes(x2[:_SPLIT], 0, 1),
        jnp.swapaxes(weight[:_SPLIT], 0, 1),
        CG_vals, p12, l_ind_M1M2, M_ptr_M1M2)
    out_tc = jnp.swapaxes(out_tct, 0, 1)
    return jnp.concatenate([out_tc, out_sc], axis=0)


# speedup vs baseline: 1.1489x; 1.1036x over previous
import functools

import jax
import jax.numpy as jnp
from jax import lax
from jax.experimental import pallas as pl
from jax.experimental.pallas import tpu as pltpu
from jax.experimental.pallas import tpu_sc as plsc

_B = 1024
_M = 16
_C = 128
_NNZ = 512
_NT = 34

_LANES = 16
_NW = 32
_CCHUNKS = _C // _LANES
_MPTR_PAD = 32

_SPLIT = 576
_BB = 64


def _sc_tensor_product(x1, x2, w, cg, p12, paw, mptr_pad):
    b_sc = x1.shape[0]
    bpw = b_sc // _NW
    mesh = plsc.VectorSubcoreMesh(core_axis_name="c", subcore_axis_name="s")

    @functools.partial(
        pl.kernel,
        mesh=mesh,
        out_type=jax.ShapeDtypeStruct((b_sc, _M, _C), jnp.float32),
        compiler_params=pltpu.CompilerParams(needs_layout_passes=False),
        scratch_types=[
            pltpu.SMEM((_NNZ,), jnp.int32),
            pltpu.SMEM((_NNZ,), jnp.int32),
            pltpu.SMEM((_NNZ,), jnp.float32),
            pltpu.SMEM((_MPTR_PAD,), jnp.int32),
            pltpu.VMEM((_NNZ,), jnp.int32),
            pltpu.VMEM((_NNZ,), jnp.int32),
            pltpu.VMEM((_NNZ,), jnp.float32),
            pltpu.VMEM((_MPTR_PAD,), jnp.int32),
            pltpu.VMEM((_M, _C), jnp.float32),
            pltpu.VMEM((_M, _C), jnp.float32),
            pltpu.VMEM((_NT, _C), jnp.float32),
            pltpu.VMEM((_M, _C), jnp.float32),
            pltpu.VMEM((_M, _C), jnp.float32),
            pltpu.VMEM((_NT, _C), jnp.float32),
            pltpu.VMEM((_M, _C), jnp.float32),
            pltpu.VMEM((_M, _C), jnp.float32),
            pltpu.SemaphoreType.DMA,
            pltpu.SemaphoreType.DMA,
            pltpu.SemaphoreType.DMA,
            pltpu.SemaphoreType.DMA,
        ],
    )
    def k(x1_hbm, x2_hbm, w_hbm, cg_hbm, p12_hbm, paw_hbm, mptr_hbm,
          out_hbm, p12_s, paw_s, cg_s, mptr_s, p12_b, paw_b, cg_b, mptr_b,
          x1_va, x2_va, w_va, x1_vb, x2_vb, w_vb, out_va, out_vb,
          sem_a, sem_b, sem_oa, sem_ob):
        wid = lax.axis_index("c") * 16 + lax.axis_index("s")

        pltpu.sync_copy(p12_hbm, p12_b)
        pltpu.sync_copy(paw_hbm, paw_b)
        pltpu.sync_copy(cg_hbm, cg_b)
        pltpu.sync_copy(mptr_hbm, mptr_b)

        @plsc.parallel_loop(0, _NNZ, _LANES)
        def fill_body(base):
            v12 = p12_b[pl.ds(base, _LANES)]
            vaw = paw_b[pl.ds(base, _LANES)]
            vcg = cg_b[pl.ds(base, _LANES)]
            for j in range(_LANES):
                p12_s[base + j] = v12[j]
                paw_s[base + j] = vaw[j]
                cg_s[base + j] = vcg[j]

        @plsc.parallel_loop(0, _MPTR_PAD, _LANES)
        def fill_mptr(base):
            vmp = mptr_b[pl.ds(base, _LANES)]
            for j in range(_LANES):
                mptr_s[base + j] = vmp[j]

        b0 = wid * bpw
        bufs = ((x1_va, x2_va, w_va, sem_a), (x1_vb, x2_vb, w_vb, sem_b))
        obufs = ((out_va, sem_oa), (out_vb, sem_ob))

        def start_copies(b, buf):
            x1d, x2d, wd, sem = buf
            pltpu.async_copy(x1_hbm.at[b], x1d, sem)
            pltpu.async_copy(x2_hbm.at[b], x2d, sem)
            pltpu.async_copy(w_hbm.at[b], wd, sem)

        def wait_copies(buf):
            x1d, x2d, wd, sem = buf
            pltpu.make_async_copy(x1_hbm.at[b0], x1d, sem).wait()
            pltpu.make_async_copy(x2_hbm.at[b0], x2d, sem).wait()
            pltpu.make_async_copy(w_hbm.at[b0], wd, sem).wait()

        def compute(b, i2, buf, obuf):
            x1d, x2d, wd, _ = buf
            out_v, sem_o = obuf

            @pl.when(i2 > 0)
            def _():
                pltpu.make_async_copy(out_v, out_hbm.at[b0], sem_o).wait()

            def seg_body(m, carry2):
                st = mptr_s[m]
                en = mptr_s[m + 1]
                zero = jnp.zeros((_LANES,), jnp.float32)
                init = (zero,) * _CCHUNKS

                @plsc.parallel_loop(st, en, 1, unroll=2, carry=init)
                def acc_fin(n, acc):
                    s12 = p12_s[n]
                    aws = paw_s[n]
                    cgs = cg_s[n]
                    o1 = s12 & 255
                    o2 = lax.shift_right_logical(s12, 8)
                    new = []
                    for kk in range(_CCHUNKS):
                        g1 = x1d[o1, pl.ds(kk * _LANES, _LANES)]
                        g2 = x2d[o2, pl.ds(kk * _LANES, _LANES)]
                        gw = wd[aws, pl.ds(kk * _LANES, _LANES)]
                        new.append(acc[kk] + g1 * g2 * gw * cgs)
                    return tuple(new)

                for kk in range(_CCHUNKS):
                    out_v[m, pl.ds(kk * _LANES, _LANES)] = acc_fin[kk]
                return carry2
            lax.fori_loop(0, _M, seg_body, 0)

            pltpu.async_copy(out_v, out_hbm.at[b], sem_o)

        start_copies(b0, bufs[0])

        def batch_pair(i2, carry):
            for par in range(2):
                i = i2 * 2 + par
                b = b0 + i
                buf = bufs[par]
                nxt = bufs[1 - par]
                wait_copies(buf)
                start_copies(b0 + jnp.minimum(i + 1, bpw - 1), nxt)
                compute(b, i2, buf, obufs[par])
            return carry
        lax.fori_loop(0, bpw // 2, batch_pair, 0)
        wait_copies(bufs[0])
        for out_v, sem_o in obufs:
            pltpu.make_async_copy(out_v, out_hbm.at[b0], sem_o).wait()

    return k(x1, x2, w, cg, p12, paw, mptr_pad)


def _tc_body(p12_ref, paw_ref, cg_ref, mptr_ref, x1_ref, x2_ref, w_ref,
             out_ref):
    for m in range(_M):
        st = mptr_ref[m]
        en = mptr_ref[m + 1]

        def e_body(n, acc):
            s12 = p12_ref[n]
            aws = paw_ref[n]
            cgs = cg_ref[n]
            o1 = s12 & 255
            o2 = lax.shift_right_logical(s12, 8)
            g1 = x1_ref[o1]
            g2 = x2_ref[o2]
            gw = w_ref[aws]
            return acc + g1 * g2 * gw * cgs

        acc = lax.fori_loop(st, en, e_body,
                            jnp.zeros((_BB, _C), jnp.float32))
        out_ref[m] = acc


def _tc_tensor_product(x1t, x2t, wt, cg, p12, paw, mptr):
    b_tc = x1t.shape[1]
    grid = (b_tc // _BB,)
    return pl.pallas_call(
        _tc_body,
        grid=grid,
        in_specs=[
            pl.BlockSpec(memory_space=pltpu.SMEM),
            pl.BlockSpec(memory_space=pltpu.SMEM),
            pl.BlockSpec(memory_space=pltpu.SMEM),
            pl.BlockSpec(memory_space=pltpu.SMEM),
            pl.BlockSpec((_M, _BB, _C), lambda i: (0, i, 0)),
            pl.BlockSpec((_M, _BB, _C), lambda i: (0, i, 0)),
            pl.BlockSpec((_NT, _BB, _C), lambda i: (0, i, 0)),
        ],
        out_specs=pl.BlockSpec((_M, _BB, _C), lambda i: (0, i, 0)),
        out_shape=jax.ShapeDtypeStruct((_M, b_tc, _C), jnp.float32),
        compiler_params=pltpu.CompilerParams(
            dimension_semantics=("arbitrary",)),
    )(p12, paw, cg, mptr, x1t, x2t, wt)


def kernel(x1, x2, weight, CG_vals, l_ind_M1M2, M1, M2, M_ptr_M1M2):
    p12 = M1 | (M2 << 8)
    mptr_pad = jnp.concatenate(
        [M_ptr_M1M2, jnp.zeros((_MPTR_PAD - _M - 1,), jnp.int32)])

    out_sc = _sc_tensor_product(
        x1[_SPLIT:], x2[_SPLIT:], weight[_SPLIT:],
        CG_vals, p12, l_ind_M1M2, mptr_pad)
    out_tct = _tc_tensor_product(
        jnp.swapaxes(x1[:_SPLIT], 0, 1),
        jnp.swapaxes(x2[:_SPLIT], 0, 1),
        jnp.swapaxes(weight[:_SPLIT], 0, 1),
        CG_vals, p12, l_ind_M1M2, M_ptr_M1M2)
    out_tc = jnp.swapaxes(out_tct, 0, 1)
    return jnp.concatenate([out_tc, out_sc], axis=0)


# speedup vs baseline: 1.3033x; 1.1344x over previous
import functools

import jax
import jax.numpy as jnp
from jax import lax
from jax.experimental import pallas as pl
from jax.experimental.pallas import tpu as pltpu
from jax.experimental.pallas import tpu_sc as plsc

_B = 1024
_M = 16
_C = 128
_NNZ = 512
_NT = 34

_LANES = 16
_NW = 32
_CCHUNKS = _C // _LANES
_MPTR_PAD = 32

_SPLIT = 640
_BB = 64


def _sc_tensor_product(x1, x2, w, cg, p12, paw, mptr_pad):
    b_sc = x1.shape[0]
    bpw = b_sc // _NW
    mesh = plsc.VectorSubcoreMesh(core_axis_name="c", subcore_axis_name="s")

    @functools.partial(
        pl.kernel,
        mesh=mesh,
        out_type=jax.ShapeDtypeStruct((b_sc, _M, _C), jnp.float32),
        compiler_params=pltpu.CompilerParams(needs_layout_passes=False),
        scratch_types=[
            pltpu.SMEM((_NNZ,), jnp.int32),
            pltpu.SMEM((_NNZ,), jnp.int32),
            pltpu.SMEM((_NNZ,), jnp.float32),
            pltpu.SMEM((_MPTR_PAD,), jnp.int32),
            pltpu.VMEM((_NNZ,), jnp.int32),
            pltpu.VMEM((_NNZ,), jnp.int32),
            pltpu.VMEM((_NNZ,), jnp.float32),
            pltpu.VMEM((_MPTR_PAD,), jnp.int32),
            pltpu.VMEM((_M, _C), jnp.float32),
            pltpu.VMEM((_M, _C), jnp.float32),
            pltpu.VMEM((_NT, _C), jnp.float32),
            pltpu.VMEM((_M, _C), jnp.float32),
            pltpu.VMEM((_M, _C), jnp.float32),
            pltpu.VMEM((_NT, _C), jnp.float32),
            pltpu.VMEM((_M, _C), jnp.float32),
            pltpu.VMEM((_M, _C), jnp.float32),
            pltpu.SemaphoreType.DMA,
            pltpu.SemaphoreType.DMA,
            pltpu.SemaphoreType.DMA,
            pltpu.SemaphoreType.DMA,
        ],
    )
    def k(x1_hbm, x2_hbm, w_hbm, cg_hbm, p12_hbm, paw_hbm, mptr_hbm,
          out_hbm, p12_s, paw_s, cg_s, mptr_s, p12_b, paw_b, cg_b, mptr_b,
          x1_va, x2_va, w_va, x1_vb, x2_vb, w_vb, out_va, out_vb,
          sem_a, sem_b, sem_oa, sem_ob):
        wid = lax.axis_index("c") * 16 + lax.axis_index("s")

        pltpu.sync_copy(p12_hbm, p12_b)
        pltpu.sync_copy(paw_hbm, paw_b)
        pltpu.sync_copy(cg_hbm, cg_b)
        pltpu.sync_copy(mptr_hbm, mptr_b)

        @plsc.parallel_loop(0, _NNZ, _LANES)
        def fill_body(base):
            v12 = p12_b[pl.ds(base, _LANES)]
            vaw = paw_b[pl.ds(base, _LANES)]
            vcg = cg_b[pl.ds(base, _LANES)]
            for j in range(_LANES):
                p12_s[base + j] = v12[j]
                paw_s[base + j] = vaw[j]
                cg_s[base + j] = vcg[j]

        @plsc.parallel_loop(0, _MPTR_PAD, _LANES)
        def fill_mptr(base):
            vmp = mptr_b[pl.ds(base, _LANES)]
            for j in range(_LANES):
                mptr_s[base + j] = vmp[j]

        b0 = wid * bpw
        bufs = ((x1_va, x2_va, w_va, sem_a), (x1_vb, x2_vb, w_vb, sem_b))
        obufs = ((out_va, sem_oa), (out_vb, sem_ob))

        def start_copies(b, buf):
            x1d, x2d, wd, sem = buf
            pltpu.async_copy(x1_hbm.at[b], x1d, sem)
            pltpu.async_copy(x2_hbm.at[b], x2d, sem)
            pltpu.async_copy(w_hbm.at[b], wd, sem)

        def wait_copies(buf):
            x1d, x2d, wd, sem = buf
            pltpu.make_async_copy(x1_hbm.at[b0], x1d, sem).wait()
            pltpu.make_async_copy(x2_hbm.at[b0], x2d, sem).wait()
            pltpu.make_async_copy(w_hbm.at[b0], wd, sem).wait()

        def compute(b, i2, buf, obuf):
            x1d, x2d, wd, _ = buf
            out_v, sem_o = obuf

            @pl.when(i2 > 0)
            def _():
                pltpu.make_async_copy(out_v, out_hbm.at[b0], sem_o).wait()

            def seg_body(m, carry2):
                st = mptr_s[m]
                en = mptr_s[m + 1]
                zero = jnp.zeros((_LANES,), jnp.float32)
                init = (zero,) * _CCHUNKS

                @plsc.parallel_loop(st, en, 1, unroll=2, carry=init)
                def acc_fin(n, acc):
                    s12 = p12_s[n]
                    aws = paw_s[n]
                    cgs = cg_s[n]
                    o1 = s12 & 255
                    o2 = lax.shift_right_logical(s12, 8)
                    new = []
                    for kk in range(_CCHUNKS):
                        g1 = x1d[o1, pl.ds(kk * _LANES, _LANES)]
                        g2 = x2d[o2, pl.ds(kk * _LANES, _LANES)]
                        gw = wd[aws, pl.ds(kk * _LANES, _LANES)]
                        new.append(acc[kk] + g1 * g2 * gw * cgs)
                    return tuple(new)

                for kk in range(_CCHUNKS):
                    out_v[m, pl.ds(kk * _LANES, _LANES)] = acc_fin[kk]
                return carry2
            lax.fori_loop(0, _M, seg_body, 0)

            pltpu.async_copy(out_v, out_hbm.at[b], sem_o)

        start_copies(b0, bufs[0])

        def batch_pair(i2, carry):
            for par in range(2):
                i = i2 * 2 + par
                b = b0 + i
                buf = bufs[par]
                nxt = bufs[1 - par]
                wait_copies(buf)
                start_copies(b0 + jnp.minimum(i + 1, bpw - 1), nxt)
                compute(b, i2, buf, obufs[par])
            return carry
        lax.fori_loop(0, bpw // 2, batch_pair, 0)
        wait_copies(bufs[0])
        for out_v, sem_o in obufs:
            pltpu.make_async_copy(out_v, out_hbm.at[b0], sem_o).wait()

    return k(x1, x2, w, cg, p12, paw, mptr_pad)


def _tc_body(p12_ref, paw_ref, cg_ref, mptr_ref, x1_ref, x2_ref, w_ref,
             out_ref):
    for m in range(_M):
        st = mptr_ref[m]
        en = mptr_ref[m + 1]

        def e_body(n, acc):
            s12 = p12_ref[n]
            aws = paw_ref[n]
            cgs = cg_ref[n]
            o1 = s12 & 255
            o2 = lax.shift_right_logical(s12, 8)
            g1 = x1_ref[o1]
            g2 = x2_ref[o2]
            gw = w_ref[aws]
            return acc + g1 * g2 * gw * cgs

        acc = lax.fori_loop(st, en, e_body,
                            jnp.zeros((_BB, _C), jnp.float32))
        out_ref[m] = acc


def _tc_tensor_product(x1t, x2t, wt, cg, p12, paw, mptr):
    b_tc = x1t.shape[1]
    grid = (b_tc // _BB,)
    return pl.pallas_call(
        _tc_body,
        grid=grid,
        in_specs=[
            pl.BlockSpec(memory_space=pltpu.SMEM),
            pl.BlockSpec(memory_space=pltpu.SMEM),
            pl.BlockSpec(memory_space=pltpu.SMEM),
            pl.BlockSpec(memory_space=pltpu.SMEM),
            pl.BlockSpec((_M, _BB, _C), lambda i: (0, i, 0)),
            pl.BlockSpec((_M, _BB, _C), lambda i: (0, i, 0)),
            pl.BlockSpec((_NT, _BB, _C), lambda i: (0, i, 0)),
        ],
        out_specs=pl.BlockSpec((_M, _BB, _C), lambda i: (0, i, 0)),
        out_shape=jax.ShapeDtypeStruct((_M, b_tc, _C), jnp.float32),
        compiler_params=pltpu.CompilerParams(
            dimension_semantics=("arbitrary",)),
    )(p12, paw, cg, mptr, x1t, x2t, wt)


def kernel(x1, x2, weight, CG_vals, l_ind_M1M2, M1, M2, M_ptr_M1M2):
    p12 = M1 | (M2 << 8)
    mptr_pad = jnp.concatenate(
        [M_ptr_M1M2, jnp.zeros((_MPTR_PAD - _M - 1,), jnp.int32)])

    out_sc = _sc_tensor_product(
        x1[_SPLIT:], x2[_SPLIT:], weight[_SPLIT:],
        CG_vals, p12, l_ind_M1M2, mptr_pad)
    out_tct = _tc_tensor_product(
        jnp.swapaxes(x1[:_SPLIT], 0, 1),
        jnp.swapaxes(x2[:_SPLIT], 0, 1),
        jnp.swapaxes(weight[:_SPLIT], 0, 1),
        CG_vals, p12, l_ind_M1M2, M_ptr_M1M2)
    out_tc = jnp.swapaxes(out_tct, 0, 1)
    return jnp.concatenate([out_tc, out_sc], axis=0)
